# baseline (speedup 1.0000x reference)
import functools

import jax
import jax.numpy as jnp
from jax import lax
from jax.experimental import pallas as pl
from jax.experimental.pallas import tpu as pltpu
from jax.experimental.pallas import tpu_sc as plsc

B = 4096
T = 200
D = 64
NC = 2
NS = 16
NW = NC * NS
ROWS = B * T
RPW = ROWS // NW
CH = 128
NCH = RPW // CH
LANES = 16
POS_REP = T + CH

_mesh = plsc.VectorSubcoreMesh(core_axis_name="c", subcore_axis_name="s")


@functools.partial(
    pl.kernel,
    mesh=_mesh,
    compiler_params=pltpu.CompilerParams(use_tc_tiling_on_sc=False),
    out_type=jax.ShapeDtypeStruct((ROWS, D), jnp.float32),
    scratch_types=[
        pltpu.VMEM((NCH, CH), jnp.int32),
        pltpu.VMEM((POS_REP, D), jnp.float32),
        pltpu.VMEM((CH, D), jnp.float32),
        pltpu.SemaphoreType.DMA,
    ],
)
def _emb_kernel(ctx_hbm, tok_hbm, pos_hbm, out_hbm, idx_v, pos_v, buf_v, sem):
    wid = lax.axis_index("s") * NC + lax.axis_index("c")
    base = wid * RPW
    pltpu.sync_copy(ctx_hbm.at[wid], idx_v)
    pltpu.sync_copy(pos_hbm, pos_v)

    def chunk_body(c, carry):
        r0 = base + c * CH
        tr0 = lax.rem(c * CH, T)
        pltpu.async_copy(tok_hbm.at[idx_v.at[c]], buf_v, sem).wait()

        def row_body(i, carry2):
            for j in range(D // LANES):
                v = pos_v[tr0 + i, pl.ds(j * LANES, LANES)]
                plsc.addupdate(buf_v.at[i, pl.ds(j * LANES, LANES)], v)
            return carry2

        lax.fori_loop(0, CH, row_body, 0, unroll=2)
        pltpu.sync_copy(buf_v, out_hbm.at[pl.ds(r0, CH)])
        return carry

    lax.fori_loop(0, NCH, chunk_body, 0)


def kernel(context, tok_table, pos_table):
    ctx = context.astype(jnp.int32).reshape(NW, NCH, CH)
    pos_rep = jnp.concatenate([pos_table, pos_table[: POS_REP - T]], axis=0)
    out = _emb_kernel(ctx, tok_table, pos_rep)
    return out.reshape(B, T, D)


# speedup vs baseline: 1.1865x; 1.1865x over previous
import functools

import jax
import jax.numpy as jnp
from jax import lax
from jax.experimental import pallas as pl
from jax.experimental.pallas import tpu as pltpu
from jax.experimental.pallas import tpu_sc as plsc

B = 4096
T = 200
D = 64
NC = 2
NS = 16
NW = NC * NS
ROWS = B * T
RPW = ROWS // NW
CH = 128
SC_ROWS = 512
GPC = SC_ROWS // CH
NSC = RPW // SC_ROWS
NCH = RPW // CH
LANES = 16
POS_REP = T + CH

_mesh = plsc.VectorSubcoreMesh(core_axis_name="c", subcore_axis_name="s")


@functools.partial(
    pl.kernel,
    mesh=_mesh,
    compiler_params=pltpu.CompilerParams(use_tc_tiling_on_sc=False),
    out_type=jax.ShapeDtypeStruct((ROWS, D), jnp.float32),
    scratch_types=[
        pltpu.VMEM((NCH, CH), jnp.int32),
        pltpu.VMEM((POS_REP, D), jnp.float32),
        pltpu.VMEM((2, SC_ROWS, D), jnp.float32),
        pltpu.SemaphoreType.DMA,
        pltpu.SemaphoreType.DMA,
        pltpu.SemaphoreType.DMA,
        pltpu.SemaphoreType.DMA,
    ],
)
def _emb_kernel(ctx_hbm, tok_hbm, pos_hbm, out_hbm, idx_v, pos_v, buf_v,
                gsem0, gsem1, ssem0, ssem1):
    gsems = (gsem0, gsem1)
    ssems = (ssem0, ssem1)
    wid = lax.axis_index("s") * NC + lax.axis_index("c")
    base = wid * RPW
    pltpu.sync_copy(ctx_hbm.at[wid], idx_v)
    pltpu.sync_copy(pos_hbm, pos_v)

    def gather_super(s, p):
        for k in range(GPC):
            pltpu.async_copy(
                tok_hbm.at[idx_v.at[s * GPC + k]],
                buf_v.at[p, pl.ds(k * CH, CH)],
                gsems[p],
            )

    def drain_gathers(p):
        pltpu.make_async_copy(
            out_hbm.at[pl.ds(0, SC_ROWS)], buf_v.at[p], gsems[p]
        ).wait()

    def wait_store(p):
        pltpu.make_async_copy(
            buf_v.at[p], out_hbm.at[pl.ds(0, SC_ROWS)], ssems[p]
        ).wait()

    def add_pos(s, p):
        for k in range(GPC):
            tr0 = lax.rem(s * SC_ROWS + k * CH, T)

            def row_body(i, carry, k=k, tr0=tr0):
                for j in range(D // LANES):
                    v = pos_v[tr0 + i, pl.ds(j * LANES, LANES)]
                    plsc.addupdate(
                        buf_v.at[p, k * CH + i, pl.ds(j * LANES, LANES)], v
                    )
                return carry

            lax.fori_loop(0, CH, row_body, 0, unroll=4)

    def start_store(s, p):
        pltpu.async_copy(
            buf_v.at[p], out_hbm.at[pl.ds(base + s * SC_ROWS, SC_ROWS)],
            ssems[p],
        )

    def step(s, p, first=False, last=False):
        q = 1 - p
        drain_gathers(p)
        if not last:
            if not first:
                wait_store(q)
            gather_super(s + 1, q)
        add_pos(s, p)
        start_store(s, p)

    gather_super(0, 0)
    step(0, 0, first=True)
    step(jnp.int32(1), 1)

    def pair_body(g, carry):
        s = g * 2
        step(s, 0)
        step(s + 1, 1)
        return carry

    lax.fori_loop(1, NSC // 2 - 1, pair_body, 0)

    step(jnp.int32(NSC - 2), 0)
    step(jnp.int32(NSC - 1), 1, last=True)
    wait_store(0)
    wait_store(1)


def kernel(context, tok_table, pos_table):
    ctx = context.astype(jnp.int32).reshape(NW, NCH, CH)
    pos_rep = jnp.concatenate([pos_table, pos_table[: POS_REP - T]], axis=0)
    out = _emb_kernel(ctx, tok_table, pos_rep)
    return out.reshape(B, T, D)


# speedup vs baseline: 1.4010x; 1.1808x over previous
import functools

import jax
import jax.numpy as jnp
from jax import lax
from jax.experimental import pallas as pl
from jax.experimental.pallas import tpu as pltpu
from jax.experimental.pallas import tpu_sc as plsc

B = 4096
T = 200
D = 64
NC = 2
NS = 16
NW = NC * NS
BPW = B // NW
BPS = 2
NSC = BPW // BPS
SC_ROWS = BPS * T
CH0 = 128
CH1 = T - CH0
LANES = 16

_mesh = plsc.VectorSubcoreMesh(core_axis_name="c", subcore_axis_name="s")


@functools.partial(
    pl.kernel,
    mesh=_mesh,
    compiler_params=pltpu.CompilerParams(use_tc_tiling_on_sc=False),
    out_type=jax.ShapeDtypeStruct((B, T, D), jnp.float32),
    scratch_types=[
        pltpu.VMEM((BPW, T), jnp.int32),
        pltpu.VMEM((T, D), jnp.float32),
        pltpu.VMEM((2, BPS, T, D), jnp.float32),
        pltpu.SemaphoreType.DMA,
        pltpu.SemaphoreType.DMA,
        pltpu.SemaphoreType.DMA,
        pltpu.SemaphoreType.DMA,
    ],
)
def _emb_kernel(ctx_hbm, tok_hbm, pos_hbm, out_hbm, idx_v, pos_v, buf_v,
                gsem0, gsem1, ssem0, ssem1):
    gsems = (gsem0, gsem1)
    ssems = (ssem0, ssem1)
    wid = lax.axis_index("s") * NC + lax.axis_index("c")
    b0 = wid * BPW
    pltpu.sync_copy(ctx_hbm.at[pl.ds(b0, BPW)], idx_v)
    pltpu.sync_copy(pos_hbm, pos_v)

    def gather_super(s, p):
        for bb in range(BPS):
            bl = s * BPS + bb
            pltpu.async_copy(
                tok_hbm.at[idx_v.at[bl, pl.ds(0, CH0)]],
                buf_v.at[p, bb, pl.ds(0, CH0)],
                gsems[p],
            )
            pltpu.async_copy(
                tok_hbm.at[idx_v.at[bl, pl.ds(CH0, CH1)]],
                buf_v.at[p, bb, pl.ds(CH0, CH1)],
                gsems[p],
            )

    def drain_gathers(p):
        pltpu.make_async_copy(
            out_hbm.at[pl.ds(0, BPS)],
            buf_v.at[p],
            gsems[p],
        ).wait()

    def wait_store(p):
        pltpu.make_async_copy(
            buf_v.at[p],
            out_hbm.at[pl.ds(0, BPS)],
            ssems[p],
        ).wait()

    def add_pos(p):
        for bb in range(BPS):

            def row_body(i, carry, bb=bb):
                for j in range(D // LANES):
                    v = pos_v[i, pl.ds(j * LANES, LANES)]
                    plsc.addupdate(
                        buf_v.at[p, bb, i, pl.ds(j * LANES, LANES)], v
                    )
                return carry

            lax.fori_loop(0, T, row_body, 0, unroll=4)

    def start_store(s, p):
        pltpu.async_copy(
            buf_v.at[p],
            out_hbm.at[pl.ds(b0 + s * BPS, BPS)],
            ssems[p],
        )

    def step(s, p, first=False, last=False):
        q = 1 - p
        drain_gathers(p)
        if not last:
            if not first:
                wait_store(q)
            gather_super(s + 1, q)
        add_pos(p)
        start_store(s, p)

    gather_super(0, 0)
    step(0, 0, first=True)
    step(jnp.int32(1), 1)

    def pair_body(g, carry):
        s = g * 2
        step(s, 0)
        step(s + 1, 1)
        return carry

    lax.fori_loop(1, NSC // 2 - 1, pair_body, 0)

    step(jnp.int32(NSC - 2), 0)
    step(jnp.int32(NSC - 1), 1, last=True)
    wait_store(0)
    wait_store(1)


def kernel(context, tok_table, pos_table):
    return _emb_kernel(context.astype(jnp.int32), tok_table, pos_table)
